# baseline (speedup 1.0000x reference)
import functools

import jax
import jax.numpy as jnp
from jax import lax
from jax.experimental import pallas as pl
from jax.experimental.pallas import tpu as pltpu
from jax.experimental.pallas import tpu_sc as plsc

E_IN = 32768
E_OUT = 8192
C = 64
F = 64
K = 8
NBKT = E_OUT * K
FW = 128

NC = 2
NS = 16
L = 16

BE = 8192
CE = E_IN // NS
OE = E_IN // (NC * NS)
RW = E_OUT // NS

_SC_CACHE = {}


def _mesh():
    return plsc.VectorSubcoreMesh(
        core_axis_name="c", subcore_axis_name="s",
        num_cores=NC, num_subcores=NS)


def _sc_counts_body(seg_hbm, kid_hbm, cnt_hbm,
                    seg_v, kid_v, comb_v, ones_v, cnt_v, zer1_v,
                    counts_sh, sem):
    cid = lax.axis_index("c")
    sid = lax.axis_index("s")
    wid = cid * NS + sid

    z16 = jnp.zeros((L,), jnp.float32)
    o16 = jnp.ones((L,), jnp.float32)

    def _z1(i, _):
        zer1_v[pl.ds(i * L, L)] = z16
        return 0
    lax.fori_loop(0, (NBKT // NS) // L, _z1, 0)
    pltpu.sync_copy(zer1_v, counts_sh.at[pl.ds(sid * (NBKT // NS), NBKT // NS)])
    plsc.subcore_barrier()

    base1 = sid * CE
    pltpu.sync_copy(seg_hbm.at[pl.ds(base1, CE)], seg_v)
    pltpu.sync_copy(kid_hbm.at[pl.ds(base1, CE)], kid_v)

    def _mk(i, _):
        sl = pl.ds(i * L, L)
        comb_v[sl] = seg_v[sl] * K + kid_v[sl]
        ones_v[sl] = o16
        return 0
    lax.fori_loop(0, CE // L, _mk, 0)

    pltpu.sync_copy(ones_v, counts_sh.at[comb_v], add=True)
    plsc.subcore_barrier()

    base2 = wid * OE
    pltpu.sync_copy(seg_hbm.at[pl.ds(base2, OE)], seg_v.at[pl.ds(0, OE)])
    pltpu.sync_copy(kid_hbm.at[pl.ds(base2, OE)], kid_v.at[pl.ds(0, OE)])

    def _mk2(i, _):
        sl = pl.ds(i * L, L)
        comb_v[sl] = seg_v[sl] * K + kid_v[sl]
        return 0
    lax.fori_loop(0, OE // L, _mk2, 0)

    pltpu.async_copy(counts_sh.at[comb_v.at[pl.ds(0, OE)]], cnt_v, sem).wait()

    def _inv(i, _):
        sl = pl.ds(i * L, L)
        cnt_v[sl] = 1.0 / jnp.maximum(cnt_v[sl], 1.0)
        return 0
    lax.fori_loop(0, OE // L, _inv, 0)
    pltpu.sync_copy(cnt_v, cnt_hbm.at[pl.ds(base2, OE)])


def _sc_counts(seg, kid):
    if "counts" not in _SC_CACHE:
        _SC_CACHE["counts"] = functools.partial(
            pl.kernel,
            out_type=jax.ShapeDtypeStruct((E_IN,), jnp.float32),
            mesh=_mesh(),
            scratch_types=[
                pltpu.VMEM((CE,), jnp.int32),
                pltpu.VMEM((CE,), jnp.int32),
                pltpu.VMEM((CE,), jnp.int32),
                pltpu.VMEM((CE,), jnp.float32),
                pltpu.VMEM((OE,), jnp.float32),
                pltpu.VMEM((NBKT // NS,), jnp.float32),
                pltpu.VMEM_SHARED((NBKT,), jnp.float32),
                pltpu.SemaphoreType.DMA,
            ],
        )(_sc_counts_body)
    return _SC_CACHE["counts"](seg, kid)


def _tap_matmul_body(feat_ref, aux_ref, dr_ref, kern_ref, y_ref):
    d = jax.nn.softplus(dr_ref[...])
    aux = jnp.transpose(aux_ref[0], (1, 0))
    dt_col = aux[:, 0:1]
    inv = aux[:, 1:2]
    kid_col = aux[:, 2:3]
    factors = jnp.exp(-dt_col * d)
    dec = feat_ref[...] * factors * inv
    yb = jnp.dot(dec, kern_ref[0],
                 preferred_element_type=jnp.float32)
    for k in range(1, K):
        yk = jnp.dot(dec, kern_ref[k], preferred_element_type=jnp.float32)
        yb = jnp.where(kid_col == float(k), yk, yb)
    y_ref[...] = jnp.concatenate(
        [yb, jnp.zeros((yb.shape[0], 128 - F), jnp.float32)], axis=1)


def _tap_matmul(features, aux3, dr2, kern3):
    nb = E_IN // BE
    return pl.pallas_call(
        _tap_matmul_body,
        grid=(nb,),
        in_specs=[
            pl.BlockSpec((BE, C), lambda i: (i, 0)),
            pl.BlockSpec((1, 3, BE), lambda i: (i, 0, 0)),
            pl.BlockSpec((1, C), lambda i: (0, 0)),
            pl.BlockSpec((K, C, F), lambda i: (0, 0, 0)),
        ],
        out_specs=pl.BlockSpec((BE, FW), lambda i: (i, 0)),
        out_shape=jax.ShapeDtypeStruct((E_IN, FW), jnp.float32),
    )(features, aux3, dr2, kern3)


YCH = 128


def _sc_segsum_body(y_hbm, seg_hbm, out_hbm,
                    s0_v, s1_v, y0_v, y1_v, zer2_v, out_sh, semA, semB):
    cid = lax.axis_index("c")
    sid = lax.axis_index("s")
    wid = cid * NS + sid

    z16 = jnp.zeros((L,), jnp.float32)

    def _z2(i, _):
        zer2_v[i // (FW // L), pl.ds((i % (FW // L)) * L, L)] = z16
        return 0
    lax.fori_loop(0, 64 * (FW // L), _z2, 0)
    for j in range(RW // 64):
        pltpu.sync_copy(zer2_v, out_sh.at[pl.ds(sid * RW + j * 64, 64), :])
    plsc.subcore_barrier()

    base2 = wid * OE

    def _pair(h, _):
        off0 = pl.multiple_of(base2 + (2 * h) * YCH, YCH)
        off1 = pl.multiple_of(base2 + (2 * h + 1) * YCH, YCH)
        cy0 = pltpu.async_copy(y_hbm.at[pl.ds(off0, YCH), :], y0_v, semA)
        cs0 = pltpu.async_copy(seg_hbm.at[pl.ds(off0, YCH)], s0_v, semA)
        cy1 = pltpu.async_copy(y_hbm.at[pl.ds(off1, YCH), :], y1_v, semB)
        cs1 = pltpu.async_copy(seg_hbm.at[pl.ds(off1, YCH)], s1_v, semB)
        cy0.wait()
        cs0.wait()
        pltpu.sync_copy(y0_v, out_sh.at[s0_v], add=True)
        cy1.wait()
        cs1.wait()
        pltpu.sync_copy(y1_v, out_sh.at[s1_v], add=True)
        return 0
    lax.fori_loop(0, OE // (2 * YCH), _pair, 0)
    plsc.subcore_barrier()

    pltpu.sync_copy(out_sh.at[pl.ds(sid * RW, RW), :],
                    out_hbm.at[cid, pl.ds(sid * RW, RW), :])


def _sc_segsum(y, seg):
    if "segsum" not in _SC_CACHE:
        _SC_CACHE["segsum"] = functools.partial(
            pl.kernel,
            out_type=jax.ShapeDtypeStruct((NC, E_OUT, FW), jnp.float32),
            mesh=_mesh(),
            scratch_types=[
                pltpu.VMEM((YCH,), jnp.int32),
                pltpu.VMEM((YCH,), jnp.int32),
                pltpu.VMEM((YCH, FW), jnp.float32),
                pltpu.VMEM((YCH, FW), jnp.float32),
                pltpu.VMEM((64, FW), jnp.float32),
                pltpu.VMEM_SHARED((E_OUT, FW), jnp.float32),
                pltpu.SemaphoreType.DMA,
                pltpu.SemaphoreType.DMA,
            ],
        )(_sc_segsum_body)
    return _SC_CACHE["segsum"](y, seg)


def _combine_body(p_ref, b_ref, o_ref):
    x = p_ref[...]
    o_ref[...] = x[0, :, :F] + x[1, :, :F] + b_ref[...]


def _combine(partials, bias2):
    bo = 2048
    return pl.pallas_call(
        _combine_body,
        grid=(E_OUT // bo,),
        in_specs=[
            pl.BlockSpec((NC, bo, FW), lambda i: (0, i, 0)),
            pl.BlockSpec((1, F), lambda i: (0, 0)),
        ],
        out_specs=pl.BlockSpec((bo, F), lambda i: (i, 0)),
        out_shape=jax.ShapeDtypeStruct((E_OUT, F), jnp.float32),
    )(partials, bias2)


def kernel(features, dt, times_out, successor_kernel_ids, segment_ids_out,
           decay_rate, kernel, bias):
    del times_out
    seg = segment_ids_out.astype(jnp.int32)
    kid = successor_kernel_ids.astype(jnp.int32)
    cnt = _sc_counts(seg, kid)
    nb = E_IN // BE
    aux3 = jnp.stack([dt.reshape(nb, BE),
                      cnt.reshape(nb, BE),
                      kid.astype(jnp.float32).reshape(nb, BE)], axis=1)
    y = _tap_matmul(features, aux3, decay_rate.reshape(1, C), kernel)
    partials = _sc_segsum(y, seg)
    return _combine(partials, bias.reshape(1, F))


# speedup vs baseline: 1.0079x; 1.0079x over previous
import functools

import jax
import jax.numpy as jnp
from jax import lax
from jax.experimental import pallas as pl
from jax.experimental.pallas import tpu as pltpu
from jax.experimental.pallas import tpu_sc as plsc

E_IN = 32768
E_OUT = 8192
C = 64
F = 64
K = 8
NBKT = E_OUT * K
FW = 128

NC = 2
NS = 16
L = 16

BE = 2048
CE = E_IN // NS
OE = E_IN // (NC * NS)
RW = E_OUT // NS

_SC_CACHE = {}


def _mesh():
    return plsc.VectorSubcoreMesh(
        core_axis_name="c", subcore_axis_name="s",
        num_cores=NC, num_subcores=NS)


def _sc_counts_body(seg_hbm, kid_hbm, cnt_hbm,
                    seg_v, kid_v, comb_v, ones_v, cnt_v, zer1_v,
                    counts_sh, sem):
    cid = lax.axis_index("c")
    sid = lax.axis_index("s")
    wid = cid * NS + sid

    z16 = jnp.zeros((L,), jnp.float32)
    o16 = jnp.ones((L,), jnp.float32)

    def _z1(i, _):
        zer1_v[pl.ds(i * L, L)] = z16
        return 0
    lax.fori_loop(0, (NBKT // NS) // L, _z1, 0)
    pltpu.sync_copy(zer1_v, counts_sh.at[pl.ds(sid * (NBKT // NS), NBKT // NS)])
    plsc.subcore_barrier()

    base1 = sid * CE
    pltpu.sync_copy(seg_hbm.at[pl.ds(base1, CE)], seg_v)
    pltpu.sync_copy(kid_hbm.at[pl.ds(base1, CE)], kid_v)

    def _mk(i, _):
        sl = pl.ds(i * L, L)
        comb_v[sl] = seg_v[sl] * K + kid_v[sl]
        ones_v[sl] = o16
        return 0
    lax.fori_loop(0, CE // L, _mk, 0)

    pltpu.sync_copy(ones_v, counts_sh.at[comb_v], add=True)
    plsc.subcore_barrier()

    base2 = wid * OE
    pltpu.sync_copy(seg_hbm.at[pl.ds(base2, OE)], seg_v.at[pl.ds(0, OE)])
    pltpu.sync_copy(kid_hbm.at[pl.ds(base2, OE)], kid_v.at[pl.ds(0, OE)])

    def _mk2(i, _):
        sl = pl.ds(i * L, L)
        comb_v[sl] = seg_v[sl] * K + kid_v[sl]
        return 0
    lax.fori_loop(0, OE // L, _mk2, 0)

    pltpu.async_copy(counts_sh.at[comb_v.at[pl.ds(0, OE)]], cnt_v, sem).wait()

    def _inv(i, _):
        sl = pl.ds(i * L, L)
        cnt_v[sl] = 1.0 / jnp.maximum(cnt_v[sl], 1.0)
        return 0
    lax.fori_loop(0, OE // L, _inv, 0)
    pltpu.sync_copy(cnt_v, cnt_hbm.at[pl.ds(base2, OE)])


def _sc_counts(seg, kid):
    if "counts" not in _SC_CACHE:
        _SC_CACHE["counts"] = functools.partial(
            pl.kernel,
            out_type=jax.ShapeDtypeStruct((E_IN,), jnp.float32),
            mesh=_mesh(),
            scratch_types=[
                pltpu.VMEM((CE,), jnp.int32),
                pltpu.VMEM((CE,), jnp.int32),
                pltpu.VMEM((CE,), jnp.int32),
                pltpu.VMEM((CE,), jnp.float32),
                pltpu.VMEM((OE,), jnp.float32),
                pltpu.VMEM((NBKT // NS,), jnp.float32),
                pltpu.VMEM_SHARED((NBKT,), jnp.float32),
                pltpu.SemaphoreType.DMA,
            ],
        )(_sc_counts_body)
    return _SC_CACHE["counts"](seg, kid)


def _tap_matmul_body(feat_ref, aux_ref, dr_ref, kern_ref, y_ref):
    d = jax.nn.softplus(dr_ref[...])
    aux = jnp.transpose(aux_ref[0], (1, 0))
    dt_col = aux[:, 0:1]
    inv = aux[:, 1:2]
    kid_col = aux[:, 2:3]
    factors = jnp.exp(-dt_col * d)
    dec = feat_ref[...] * factors * inv
    yb = jnp.dot(dec, kern_ref[0],
                 preferred_element_type=jnp.float32)
    for k in range(1, K):
        yk = jnp.dot(dec, kern_ref[k], preferred_element_type=jnp.float32)
        yb = jnp.where(kid_col == float(k), yk, yb)
    y_ref[...] = jnp.concatenate(
        [yb, jnp.zeros((yb.shape[0], 128 - F), jnp.float32)], axis=1)


def _tap_matmul(features, aux3, dr2, kern3):
    nb = E_IN // BE
    return pl.pallas_call(
        _tap_matmul_body,
        grid=(nb,),
        in_specs=[
            pl.BlockSpec((BE, C), lambda i: (i, 0)),
            pl.BlockSpec((1, 3, BE), lambda i: (i, 0, 0)),
            pl.BlockSpec((1, C), lambda i: (0, 0)),
            pl.BlockSpec((K, C, F), lambda i: (0, 0, 0)),
        ],
        out_specs=pl.BlockSpec((BE, FW), lambda i: (i, 0)),
        out_shape=jax.ShapeDtypeStruct((E_IN, FW), jnp.float32),
    )(features, aux3, dr2, kern3)


YCH = 128


def _sc_segsum_body(y_hbm, seg_hbm, out_hbm,
                    s0_v, s1_v, y0_v, y1_v, zer2_v, out_sh, semA, semB):
    cid = lax.axis_index("c")
    sid = lax.axis_index("s")
    wid = cid * NS + sid

    z16 = jnp.zeros((L,), jnp.float32)

    def _z2(i, _):
        zer2_v[i // (FW // L), pl.ds((i % (FW // L)) * L, L)] = z16
        return 0
    lax.fori_loop(0, 64 * (FW // L), _z2, 0)
    for j in range(RW // 64):
        pltpu.sync_copy(zer2_v, out_sh.at[pl.ds(sid * RW + j * 64, 64), :])
    plsc.subcore_barrier()

    base2 = wid * OE

    def _pair(h, _):
        off0 = pl.multiple_of(base2 + (2 * h) * YCH, YCH)
        off1 = pl.multiple_of(base2 + (2 * h + 1) * YCH, YCH)
        cy0 = pltpu.async_copy(y_hbm.at[pl.ds(off0, YCH), :], y0_v, semA)
        cs0 = pltpu.async_copy(seg_hbm.at[pl.ds(off0, YCH)], s0_v, semA)
        cy1 = pltpu.async_copy(y_hbm.at[pl.ds(off1, YCH), :], y1_v, semB)
        cs1 = pltpu.async_copy(seg_hbm.at[pl.ds(off1, YCH)], s1_v, semB)
        cy0.wait()
        cs0.wait()
        pltpu.sync_copy(y0_v, out_sh.at[s0_v], add=True)
        cy1.wait()
        cs1.wait()
        pltpu.sync_copy(y1_v, out_sh.at[s1_v], add=True)
        return 0
    lax.fori_loop(0, OE // (2 * YCH), _pair, 0)
    plsc.subcore_barrier()

    pltpu.sync_copy(out_sh.at[pl.ds(sid * RW, RW), :],
                    out_hbm.at[cid, pl.ds(sid * RW, RW), :])


def _sc_segsum(y, seg):
    if "segsum" not in _SC_CACHE:
        _SC_CACHE["segsum"] = functools.partial(
            pl.kernel,
            out_type=jax.ShapeDtypeStruct((NC, E_OUT, FW), jnp.float32),
            mesh=_mesh(),
            scratch_types=[
                pltpu.VMEM((YCH,), jnp.int32),
                pltpu.VMEM((YCH,), jnp.int32),
                pltpu.VMEM((YCH, FW), jnp.float32),
                pltpu.VMEM((YCH, FW), jnp.float32),
                pltpu.VMEM((64, FW), jnp.float32),
                pltpu.VMEM_SHARED((E_OUT, FW), jnp.float32),
                pltpu.SemaphoreType.DMA,
                pltpu.SemaphoreType.DMA,
            ],
        )(_sc_segsum_body)
    return _SC_CACHE["segsum"](y, seg)


def _combine_body(p_ref, b_ref, o_ref):
    x = p_ref[...]
    o_ref[...] = x[0, :, :F] + x[1, :, :F] + b_ref[...]


def _combine(partials, bias2):
    bo = 2048
    return pl.pallas_call(
        _combine_body,
        grid=(E_OUT // bo,),
        in_specs=[
            pl.BlockSpec((NC, bo, FW), lambda i: (0, i, 0)),
            pl.BlockSpec((1, F), lambda i: (0, 0)),
        ],
        out_specs=pl.BlockSpec((bo, F), lambda i: (i, 0)),
        out_shape=jax.ShapeDtypeStruct((E_OUT, F), jnp.float32),
    )(partials, bias2)


def kernel(features, dt, times_out, successor_kernel_ids, segment_ids_out,
           decay_rate, kernel, bias):
    del times_out
    seg = segment_ids_out.astype(jnp.int32)
    kid = successor_kernel_ids.astype(jnp.int32)
    cnt = _sc_counts(seg, kid)
    nb = E_IN // BE
    aux3 = jnp.stack([dt.reshape(nb, BE),
                      cnt.reshape(nb, BE),
                      kid.astype(jnp.float32).reshape(nb, BE)], axis=1)
    y = _tap_matmul(features, aux3, decay_rate.reshape(1, C), kernel)
    partials = _sc_segsum(y, seg)
    return _combine(partials, bias.reshape(1, F))


# speedup vs baseline: 1.0217x; 1.0137x over previous
import functools

import jax
import jax.numpy as jnp
from jax import lax
from jax.experimental import pallas as pl
from jax.experimental.pallas import tpu as pltpu
from jax.experimental.pallas import tpu_sc as plsc

E_IN = 32768
E_OUT = 8192
C = 64
F = 64
K = 8
NBKT = E_OUT * K
FW = 128

NC = 2
NS = 16
L = 16

BE = 4096
CE = E_IN // NS
OE = E_IN // (NC * NS)
RW = E_OUT // NS

_SC_CACHE = {}


def _mesh():
    return plsc.VectorSubcoreMesh(
        core_axis_name="c", subcore_axis_name="s",
        num_cores=NC, num_subcores=NS)


def _sc_counts_body(seg_hbm, kid_hbm, cnt_hbm,
                    seg_v, kid_v, comb_v, ones_v, cnt_v, zer1_v,
                    counts_sh, sem):
    cid = lax.axis_index("c")
    sid = lax.axis_index("s")
    wid = cid * NS + sid

    z16 = jnp.zeros((L,), jnp.float32)
    o16 = jnp.ones((L,), jnp.float32)

    def _z1(i, _):
        zer1_v[pl.ds(i * L, L)] = z16
        return 0
    lax.fori_loop(0, (NBKT // NS) // L, _z1, 0)
    pltpu.sync_copy(zer1_v, counts_sh.at[pl.ds(sid * (NBKT // NS), NBKT // NS)])
    plsc.subcore_barrier()

    base1 = sid * CE
    pltpu.sync_copy(seg_hbm.at[pl.ds(base1, CE)], seg_v)
    pltpu.sync_copy(kid_hbm.at[pl.ds(base1, CE)], kid_v)

    def _mk(i, _):
        sl = pl.ds(i * L, L)
        comb_v[sl] = seg_v[sl] * K + kid_v[sl]
        ones_v[sl] = o16
        return 0
    lax.fori_loop(0, CE // L, _mk, 0)

    pltpu.sync_copy(ones_v, counts_sh.at[comb_v], add=True)
    plsc.subcore_barrier()

    base2 = wid * OE
    pltpu.sync_copy(seg_hbm.at[pl.ds(base2, OE)], seg_v.at[pl.ds(0, OE)])
    pltpu.sync_copy(kid_hbm.at[pl.ds(base2, OE)], kid_v.at[pl.ds(0, OE)])

    def _mk2(i, _):
        sl = pl.ds(i * L, L)
        comb_v[sl] = seg_v[sl] * K + kid_v[sl]
        return 0
    lax.fori_loop(0, OE // L, _mk2, 0)

    pltpu.async_copy(counts_sh.at[comb_v.at[pl.ds(0, OE)]], cnt_v, sem).wait()

    def _inv(i, _):
        sl = pl.ds(i * L, L)
        cnt_v[sl] = 1.0 / jnp.maximum(cnt_v[sl], 1.0)
        return 0
    lax.fori_loop(0, OE // L, _inv, 0)
    pltpu.sync_copy(cnt_v, cnt_hbm.at[pl.ds(base2, OE)])


def _sc_counts(seg, kid):
    if "counts" not in _SC_CACHE:
        _SC_CACHE["counts"] = functools.partial(
            pl.kernel,
            out_type=jax.ShapeDtypeStruct((E_IN,), jnp.float32),
            mesh=_mesh(),
            scratch_types=[
                pltpu.VMEM((CE,), jnp.int32),
                pltpu.VMEM((CE,), jnp.int32),
                pltpu.VMEM((CE,), jnp.int32),
                pltpu.VMEM((CE,), jnp.float32),
                pltpu.VMEM((OE,), jnp.float32),
                pltpu.VMEM((NBKT // NS,), jnp.float32),
                pltpu.VMEM_SHARED((NBKT,), jnp.float32),
                pltpu.SemaphoreType.DMA,
            ],
        )(_sc_counts_body)
    return _SC_CACHE["counts"](seg, kid)


def _tap_matmul_body(feat_ref, aux_ref, dr_ref, kern_ref, y_ref):
    d = jax.nn.softplus(dr_ref[...])
    aux = jnp.transpose(aux_ref[0], (1, 0))
    dt_col = aux[:, 0:1]
    inv = aux[:, 1:2]
    kid_col = aux[:, 2:3]
    factors = jnp.exp(-dt_col * d)
    dec = feat_ref[...] * factors * inv
    yb = jnp.dot(dec, kern_ref[0],
                 preferred_element_type=jnp.float32)
    for k in range(1, K):
        yk = jnp.dot(dec, kern_ref[k], preferred_element_type=jnp.float32)
        yb = jnp.where(kid_col == float(k), yk, yb)
    y_ref[...] = jnp.concatenate(
        [yb, jnp.zeros((yb.shape[0], 128 - F), jnp.float32)], axis=1)


def _tap_matmul(features, aux3, dr2, kern3):
    nb = E_IN // BE
    return pl.pallas_call(
        _tap_matmul_body,
        grid=(nb,),
        in_specs=[
            pl.BlockSpec((BE, C), lambda i: (i, 0)),
            pl.BlockSpec((1, 3, BE), lambda i: (i, 0, 0)),
            pl.BlockSpec((1, C), lambda i: (0, 0)),
            pl.BlockSpec((K, C, F), lambda i: (0, 0, 0)),
        ],
        out_specs=pl.BlockSpec((BE, FW), lambda i: (i, 0)),
        out_shape=jax.ShapeDtypeStruct((E_IN, FW), jnp.float32),
    )(features, aux3, dr2, kern3)


YCH = 128


def _sc_segsum_body(y_hbm, seg_hbm, out_hbm,
                    s0_v, s1_v, y0_v, y1_v, zer2_v, out_sh, semA, semB):
    cid = lax.axis_index("c")
    sid = lax.axis_index("s")
    wid = cid * NS + sid

    z16 = jnp.zeros((L,), jnp.float32)

    def _z2(i, _):
        zer2_v[i // (FW // L), pl.ds((i % (FW // L)) * L, L)] = z16
        return 0
    lax.fori_loop(0, 64 * (FW // L), _z2, 0)
    for j in range(RW // 64):
        pltpu.sync_copy(zer2_v, out_sh.at[pl.ds(sid * RW + j * 64, 64), :])
    plsc.subcore_barrier()

    base2 = wid * OE

    def _pair(h, _):
        off0 = pl.multiple_of(base2 + (2 * h) * YCH, YCH)
        off1 = pl.multiple_of(base2 + (2 * h + 1) * YCH, YCH)
        cy0 = pltpu.async_copy(y_hbm.at[pl.ds(off0, YCH), :], y0_v, semA)
        cs0 = pltpu.async_copy(seg_hbm.at[pl.ds(off0, YCH)], s0_v, semA)
        cy1 = pltpu.async_copy(y_hbm.at[pl.ds(off1, YCH), :], y1_v, semB)
        cs1 = pltpu.async_copy(seg_hbm.at[pl.ds(off1, YCH)], s1_v, semB)
        cy0.wait()
        cs0.wait()
        pltpu.sync_copy(y0_v, out_sh.at[s0_v], add=True)
        cy1.wait()
        cs1.wait()
        pltpu.sync_copy(y1_v, out_sh.at[s1_v], add=True)
        return 0
    lax.fori_loop(0, OE // (2 * YCH), _pair, 0)
    plsc.subcore_barrier()

    pltpu.sync_copy(out_sh.at[pl.ds(sid * RW, RW), :],
                    out_hbm.at[cid, pl.ds(sid * RW, RW), :])


def _sc_segsum(y, seg):
    if "segsum" not in _SC_CACHE:
        _SC_CACHE["segsum"] = functools.partial(
            pl.kernel,
            out_type=jax.ShapeDtypeStruct((NC, E_OUT, FW), jnp.float32),
            mesh=_mesh(),
            scratch_types=[
                pltpu.VMEM((YCH,), jnp.int32),
                pltpu.VMEM((YCH,), jnp.int32),
                pltpu.VMEM((YCH, FW), jnp.float32),
                pltpu.VMEM((YCH, FW), jnp.float32),
                pltpu.VMEM((64, FW), jnp.float32),
                pltpu.VMEM_SHARED((E_OUT, FW), jnp.float32),
                pltpu.SemaphoreType.DMA,
                pltpu.SemaphoreType.DMA,
            ],
        )(_sc_segsum_body)
    return _SC_CACHE["segsum"](y, seg)


def _combine_body(p_ref, b_ref, o_ref):
    x = p_ref[...]
    o_ref[...] = x[0, :, :F] + x[1, :, :F] + b_ref[...]


def _combine(partials, bias2):
    bo = 2048
    return pl.pallas_call(
        _combine_body,
        grid=(E_OUT // bo,),
        in_specs=[
            pl.BlockSpec((NC, bo, FW), lambda i: (0, i, 0)),
            pl.BlockSpec((1, F), lambda i: (0, 0)),
        ],
        out_specs=pl.BlockSpec((bo, F), lambda i: (i, 0)),
        out_shape=jax.ShapeDtypeStruct((E_OUT, F), jnp.float32),
    )(partials, bias2)


def kernel(features, dt, times_out, successor_kernel_ids, segment_ids_out,
           decay_rate, kernel, bias):
    del times_out
    seg = segment_ids_out.astype(jnp.int32)
    kid = successor_kernel_ids.astype(jnp.int32)
    cnt = _sc_counts(seg, kid)
    nb = E_IN // BE
    aux3 = jnp.stack([dt.reshape(nb, BE),
                      cnt.reshape(nb, BE),
                      kid.astype(jnp.float32).reshape(nb, BE)], axis=1)
    y = _tap_matmul(features, aux3, decay_rate.reshape(1, C), kernel)
    partials = _sc_segsum(y, seg)
    return _combine(partials, bias.reshape(1, F))
